# Initial kernel scaffold; baseline (speedup 1.0000x reference)
import jax
import jax.numpy as jnp
from jax.experimental import pallas as pl


def kernel(x, We, be, ln1_s, ln1_b, W1, b1, W2, b2, ln2_s, ln2_b):
    raise NotImplementedError("write your pallas kernel here")


# speedup vs baseline: 1.9656x; 1.9656x over previous
import math

import jax
import jax.numpy as jnp
import numpy as np
from jax.experimental import pallas as pl
from jax.experimental.pallas import tpu as pltpu

_NHEAD = 16
_NLAYERS = 3
_FACTOR = 5
_L = 2048
_DH = 64
_HID = 1024
_DFF = 4096
_NTOP = _FACTOR * int(np.ceil(np.log(_L)))
_HIGHEST = jax.lax.Precision.HIGHEST


def _pos_encoding_np(d_model, max_len=5000):
    position = np.arange(max_len, dtype=np.float32)[:, None]
    div_term = np.exp(
        np.arange(0, d_model, 2, dtype=np.float32) * (-math.log(10000.0) / d_model)
    )
    pe = np.zeros((max_len, d_model), dtype=np.float32)
    pe[:, 0::2] = np.sin(position * div_term)
    pe[:, 1::2] = np.cos(position * div_term)
    return pe


_PE = _pos_encoding_np(_HID)[:_L]


def _sample_constants():
    cnts, negs = [], []
    cpu = jax.local_devices(backend="cpu")[0]
    with jax.default_device(cpu):
        base = jax.random.key(42)
        for i in range(_NLAYERS):
            k = jax.random.fold_in(base, i)
            idx = np.asarray(
                jax.random.randint(k, (_L, _FACTOR * 8), 0, _L), dtype=np.int64
            )
            cnt = np.zeros((_L, _L), dtype=np.float32)
            np.add.at(cnt, (np.arange(_L)[:, None], idx), 1.0)
            neg = np.where(cnt > 0.0, 0.0, -1e30).astype(np.float32)
            cnts.append(np.ascontiguousarray(cnt.T))
            negs.append(np.ascontiguousarray(neg.T))
    return cnts, negs


_CNT_T, _NEG_T = _sample_constants()


def _dot(a, b, dims, prec=_HIGHEST):
    return jax.lax.dot_general(
        a, b, dimension_numbers=(dims, ((), ())),
        precision=prec, preferred_element_type=jnp.float32,
    )


def _embed_kernel(x_ref, we_ref, be_ref, pe_ref, o_ref):
    h = _dot(x_ref[...], we_ref[...], (((1,), (1,))))
    o_ref[...] = h + be_ref[...] + pe_ref[...]


def _m_kernel(q_ref, k_ref, neg_ref, cnt_ref, m_ref):
    s = _dot(k_ref[0], q_ref[0], (((1,), (1,))))
    ms = jnp.max(s + neg_ref[...], axis=0)
    ssum = jnp.sum(s * cnt_ref[...], axis=0)
    m_ref[0, 0, 0, :] = ms - ssum * (1.0 / _L)


def _topk_kernel(m_ref, o_ref):
    mc = m_ref[...]
    iota = jax.lax.broadcasted_iota(jnp.int32, (_NHEAD, _L), 1)
    acc_iota = jax.lax.broadcasted_iota(jnp.int32, (_NHEAD, 128), 1)

    def body(u, carry):
        mcur, acc = carry
        mval = jnp.max(mcur, axis=1, keepdims=True)
        sel = mcur == mval
        idxv = jnp.min(jnp.where(sel, iota, _L), axis=1, keepdims=True)
        acc = jnp.where(acc_iota == u, idxv, acc)
        mcur = jnp.where(iota == idxv, -jnp.inf, mcur)
        return mcur, acc

    _, acc = jax.lax.fori_loop(
        0, _NTOP, body, (mc, jnp.full((_NHEAD, 128), -1, jnp.int32))
    )
    o_ref[:, 0, :] = acc


def _attn_kernel(h_ref, ind_ref, o_ref):
    q = h_ref[0]
    ind = ind_ref[0][:, :64]
    rows = jax.lax.broadcasted_iota(jnp.int32, (_L, 64), 0)
    p = (rows == ind).astype(jnp.float32)
    qr = _dot(p, q, (((0,), (0,))))
    scores = _dot(qr, q, (((1,), (1,)))) * (1.0 / math.sqrt(_DH))
    mx = jnp.max(scores, axis=1, keepdims=True)
    e = jnp.exp(scores - mx)
    sm = e / jnp.sum(e, axis=1, keepdims=True)
    upd = _dot(sm, q, (((1,), (0,))))
    meanv = jnp.mean(q, axis=0, keepdims=True)
    o_ref[0, :, :] = meanv + _dot(p, upd - meanv, (((1,), (0,))))


def _erf(a):
    s = jnp.sign(a)
    x = jnp.abs(a)
    t = 1.0 / (1.0 + 0.3275911 * x)
    p = ((((1.061405429 * t - 1.453152027) * t + 1.421413741) * t
          - 0.284496736) * t + 0.254829592) * t
    return s * (1.0 - p * jnp.exp(-x * x))


def _gelu_exact(x):
    return 0.5 * x * (1.0 + _erf(x * np.float32(1.0 / math.sqrt(2.0))))


def _layer_norm(x, scale, bias):
    mu = jnp.mean(x, axis=1, keepdims=True)
    var = jnp.mean((x - mu) ** 2, axis=1, keepdims=True)
    return (x - mu) / jnp.sqrt(var + 1e-5) * scale + bias


def _ffn_kernel(h_ref, nx_ref, l1s_ref, l1b_ref, w1_ref, b1_ref, w2_ref,
                b2_ref, l2s_ref, l2b_ref, o_ref, xln_ref, acc_ref):
    k = pl.program_id(1)
    nk = pl.num_programs(1)

    @pl.when(k == 0)
    def _():
        x = h_ref[...] + nx_ref[...]
        xln_ref[...] = _layer_norm(x, l1s_ref[...], l1b_ref[...])
        acc_ref[...] = jnp.zeros_like(acc_ref)

    xln = xln_ref[...]
    y = _dot(xln, w1_ref[...], (((1,), (1,)))) + b1_ref[...]
    y = _gelu_exact(y)
    acc_ref[...] += _dot(y, w2_ref[...], (((1,), (1,))))

    @pl.when(k == nk - 1)
    def _():
        x2 = xln_ref[...] + acc_ref[...] + b2_ref[...]
        o_ref[...] = _layer_norm(x2, l2s_ref[...], l2b_ref[...])


_F32 = jnp.float32


def _embed(x2, we, be):
    return pl.pallas_call(
        _embed_kernel,
        out_shape=jax.ShapeDtypeStruct((_L, _HID), _F32),
    )(x2, we, be.reshape(1, _HID), jnp.asarray(_PE))


def _m_scores(h4, neg_t, cnt_t):
    rb, nr = 256, _L // 256
    return pl.pallas_call(
        _m_kernel,
        grid=(nr, _NHEAD),
        in_specs=[
            pl.BlockSpec((1, rb, _DH), lambda r, hh: (hh, r, 0)),
            pl.BlockSpec((1, _L, _DH), lambda r, hh: (hh, 0, 0)),
            pl.BlockSpec((_L, rb), lambda r, hh: (0, r)),
            pl.BlockSpec((_L, rb), lambda r, hh: (0, r)),
        ],
        out_specs=pl.BlockSpec((1, 1, 1, rb), lambda r, hh: (hh, r, 0, 0)),
        out_shape=jax.ShapeDtypeStruct((_NHEAD, nr, 1, rb), _F32),
        compiler_params=pltpu.CompilerParams(
            dimension_semantics=("arbitrary", "arbitrary")
        ),
    )(h4, h4, neg_t, cnt_t)


def _topk(m):
    return pl.pallas_call(
        _topk_kernel,
        out_shape=jax.ShapeDtypeStruct((_NHEAD, 1, 128), jnp.int32),
    )(m)


def _attn(h4, ind):
    return pl.pallas_call(
        _attn_kernel,
        grid=(_NHEAD,),
        in_specs=[
            pl.BlockSpec((1, _L, _DH), lambda hh: (hh, 0, 0)),
            pl.BlockSpec((1, 1, 128), lambda hh: (hh, 0, 0)),
        ],
        out_specs=pl.BlockSpec((1, _L, _DH), lambda hh: (hh, 0, 0)),
        out_shape=jax.ShapeDtypeStruct((_NHEAD, _L, _DH), _F32),
        compiler_params=pltpu.CompilerParams(
            dimension_semantics=("arbitrary",)
        ),
    )(h4, ind)


def _ffn(h, nx, l1s, l1b, w1, b1, w2, b2, l2s, l2b):
    rb, nr = 256, _L // 256
    kb, nkk = 512, _DFF // 512
    vec = lambda: pl.BlockSpec((1, _HID), lambda r, k: (0, 0))
    return pl.pallas_call(
        _ffn_kernel,
        grid=(nr, nkk),
        in_specs=[
            pl.BlockSpec((rb, _HID), lambda r, k: (r, 0)),
            pl.BlockSpec((rb, _HID), lambda r, k: (r, 0)),
            vec(), vec(),
            pl.BlockSpec((kb, _HID), lambda r, k: (k, 0)),
            pl.BlockSpec((1, kb), lambda r, k: (0, k)),
            pl.BlockSpec((_HID, kb), lambda r, k: (0, k)),
            vec(),
            vec(), vec(),
        ],
        out_specs=pl.BlockSpec((rb, _HID), lambda r, k: (r, 0)),
        out_shape=jax.ShapeDtypeStruct((_L, _HID), _F32),
        scratch_shapes=[
            pltpu.VMEM((rb, _HID), _F32),
            pltpu.VMEM((rb, _HID), _F32),
        ],
        compiler_params=pltpu.CompilerParams(
            dimension_semantics=("arbitrary", "arbitrary")
        ),
    )(h, nx, l1s.reshape(1, _HID), l1b.reshape(1, _HID), w1,
      b1.reshape(1, _DFF), w2, b2.reshape(1, _HID),
      l2s.reshape(1, _HID), l2b.reshape(1, _HID))


def kernel(x, We, be, ln1_s, ln1_b, W1, b1, W2, b2, ln2_s, ln2_b):
    B = x.shape[0]
    h = _embed(x.reshape(B * _L, -1), We, be)
    for i in range(_NLAYERS):
        h4 = h.reshape(_L, _NHEAD, _DH).transpose(1, 0, 2)
        m = _m_scores(h4, jnp.asarray(_NEG_T[i]), jnp.asarray(_CNT_T[i]))
        ind = _topk(m.reshape(_NHEAD, _L))
        nx = _attn(h4, ind).transpose(1, 0, 2).reshape(_L, _HID)
        h = _ffn(h, nx, ln1_s[i], ln1_b[i], W1[i], b1[i], W2[i], b2[i],
                 ln2_s[i], ln2_b[i])
    return h.reshape(B, _L, _HID)


# speedup vs baseline: 3.2179x; 1.6371x over previous
import math

import jax
import jax.numpy as jnp
import numpy as np
from jax.experimental import pallas as pl
from jax.experimental.pallas import tpu as pltpu

_NHEAD = 16
_NLAYERS = 3
_FACTOR = 5
_L = 2048
_DH = 64
_HID = 1024
_DFF = 4096
_NTOP = _FACTOR * int(np.ceil(np.log(_L)))
_HIGHEST = jax.lax.Precision.HIGHEST


def _pos_encoding_np(d_model, max_len=5000):
    position = np.arange(max_len, dtype=np.float32)[:, None]
    div_term = np.exp(
        np.arange(0, d_model, 2, dtype=np.float32) * (-math.log(10000.0) / d_model)
    )
    pe = np.zeros((max_len, d_model), dtype=np.float32)
    pe[:, 0::2] = np.sin(position * div_term)
    pe[:, 1::2] = np.cos(position * div_term)
    return pe


_PE = _pos_encoding_np(_HID)[:_L]


def _rotl(x, d):
    return ((x << np.uint32(d)) | (x >> np.uint32(32 - d))).astype(np.uint32)


def _threefry2x32(k0, k1, x0, x1):
    x0 = x0.astype(np.uint32).copy()
    x1 = x1.astype(np.uint32).copy()
    ks0, ks1 = np.uint32(k0), np.uint32(k1)
    ks2 = np.uint32(ks0 ^ ks1 ^ np.uint32(0x1BD11BDA))
    ks = [ks0, ks1, ks2]
    rot = [[13, 15, 26, 6], [17, 29, 16, 24]]
    x0 += ks0
    x1 += ks1
    for r in range(5):
        for d in rot[r % 2]:
            x0 += x1
            x1 = _rotl(x1, d)
            x1 ^= x0
        x0 += ks[(r + 1) % 3]
        x1 += ks[(r + 2) % 3] + np.uint32(r + 1)
    return x0, x1


def _tf_bits(key, n):
    iot = np.arange(n, dtype=np.uint64)
    x0 = (iot >> np.uint64(32)).astype(np.uint32)
    x1 = (iot & np.uint64(0xFFFFFFFF)).astype(np.uint32)
    o0, o1 = _threefry2x32(key[0], key[1], x0, x1)
    return o0 ^ o1


def _tf_fold_in(key, data):
    d0 = np.uint32((int(data) >> 32) & 0xFFFFFFFF)
    d1 = np.uint32(int(data) & 0xFFFFFFFF)
    o0, o1 = _threefry2x32(key[0], key[1], np.array([d0]), np.array([d1]))
    return (o0[0], o1[0])


def _tf_randint(key, n, span):
    o0, o1 = _threefry2x32(key[0], key[1], np.zeros(2, np.uint32),
                           np.arange(2, dtype=np.uint32))
    hi = _tf_bits((o0[0], o1[0]), n)
    lo = _tf_bits((o0[1], o1[1]), n)
    sp = np.uint32(span)
    mult = np.uint32(((2 ** 16) % span) * ((2 ** 16) % span) % span)
    return ((hi % sp) * mult + lo % sp) % sp


def _sample_constants():
    cnts, negs = [], []
    base = (np.uint32(0), np.uint32(42))
    for i in range(_NLAYERS):
        k = _tf_fold_in(base, i)
        idx = _tf_randint(k, _L * _FACTOR * 8, _L).astype(np.int64)
        idx = idx.reshape(_L, _FACTOR * 8)
        cnt = np.zeros((_L, _L), dtype=np.float32)
        np.add.at(cnt, (np.arange(_L)[:, None], idx), 1.0)
        neg = np.where(cnt > 0.0, 0.0, -1e30).astype(np.float32)
        cnts.append(cnt)
        negs.append(np.ascontiguousarray(neg.T))
    return cnts, negs


_CNT, _NEG_T = _sample_constants()


_DEFAULT = jax.lax.Precision.DEFAULT


def _dot(a, b, dims, prec=_HIGHEST):
    return jax.lax.dot_general(
        a, b, dimension_numbers=(dims, ((), ())),
        precision=prec, preferred_element_type=jnp.float32,
    )


def _dot3(a, b, dims):
    ahi = a.astype(jnp.bfloat16).astype(jnp.float32)
    alo = (a - ahi).astype(jnp.bfloat16)
    bhi = b.astype(jnp.bfloat16).astype(jnp.float32)
    blo = (b - bhi).astype(jnp.bfloat16)
    ah = ahi.astype(jnp.bfloat16)
    bh = bhi.astype(jnp.bfloat16)
    return (_dot(ah, bh, dims, _DEFAULT)
            + _dot(ah, blo, dims, _DEFAULT)
            + _dot(alo, bh, dims, _DEFAULT))


def _embed_kernel(x_ref, we_ref, be_ref, pe_ref, o_ref):
    h = _dot(x_ref[...], we_ref[...], (((1,), (1,))), _DEFAULT)
    o_ref[...] = h + be_ref[...] + pe_ref[...]


def _term_kernel(cnt_ref, hfull_ref, hblk_ref, t_ref):
    ks = _dot(cnt_ref[...], hfull_ref[...], (((1,), (0,))), _DEFAULT)
    prod = ks * hblk_ref[...]
    rb = prod.shape[0]
    t_ref[...] = jnp.sum(prod.reshape(rb, _NHEAD, _DH), axis=2)


def _m_kernel(q_ref, k_ref, neg_ref, m_ref):
    s = _dot3(k_ref[0], q_ref[0], (((1,), (1,))))
    m_ref[0, 0, 0, :] = jnp.max(s + neg_ref[...], axis=0)


def _topk_kernel(m_ref, t_ref, o_ref):
    mc = m_ref[...] - t_ref[...] * (1.0 / _L)
    iota = jax.lax.broadcasted_iota(jnp.int32, (_NHEAD, _L), 1)
    acc_iota = jax.lax.broadcasted_iota(jnp.int32, (_NHEAD, 128), 1)

    def body(u, carry):
        mcur, acc = carry
        mval = jnp.max(mcur, axis=1, keepdims=True)
        sel = mcur == mval
        idxv = jnp.min(jnp.where(sel, iota, _L), axis=1, keepdims=True)
        acc = jnp.where(acc_iota == u, idxv, acc)
        mcur = jnp.where(iota == idxv, -jnp.inf, mcur)
        return mcur, acc

    _, acc = jax.lax.fori_loop(
        0, _NTOP, body, (mc, jnp.full((_NHEAD, 128), -1, jnp.int32))
    )
    o_ref[:, 0, :] = acc


def _attn_kernel(h_ref, ind_ref, o_ref):
    q = h_ref[0]
    ind = ind_ref[0][:, :64]
    rows = jax.lax.broadcasted_iota(jnp.int32, (_L, 64), 0)
    p = (rows == ind).astype(jnp.float32)
    qr = _dot(p, q, (((0,), (0,))))
    scores = _dot(qr, q, (((1,), (1,)))) * (1.0 / math.sqrt(_DH))
    mx = jnp.max(scores, axis=1, keepdims=True)
    e = jnp.exp(scores - mx)
    sm = e / jnp.sum(e, axis=1, keepdims=True)
    upd = _dot(sm, q, (((1,), (0,))))
    meanv = jnp.mean(q, axis=0, keepdims=True)
    o_ref[0, :, :] = meanv + _dot(p, upd - meanv, (((1,), (0,))))


def _erf(a):
    s = jnp.sign(a)
    x = jnp.abs(a)
    t = 1.0 / (1.0 + 0.3275911 * x)
    p = ((((1.061405429 * t - 1.453152027) * t + 1.421413741) * t
          - 0.284496736) * t + 0.254829592) * t
    return s * (1.0 - p * jnp.exp(-x * x))


def _gelu_exact(x):
    return 0.5 * x * (1.0 + _erf(x * np.float32(1.0 / math.sqrt(2.0))))


def _layer_norm(x, scale, bias):
    mu = jnp.mean(x, axis=1, keepdims=True)
    var = jnp.mean((x - mu) ** 2, axis=1, keepdims=True)
    return (x - mu) / jnp.sqrt(var + 1e-5) * scale + bias


def _ffn_kernel(h_ref, nx_ref, l1s_ref, l1b_ref, w1_ref, b1_ref, w2_ref,
                b2_ref, l2s_ref, l2b_ref, o_ref, xln_ref, acc_ref):
    k = pl.program_id(1)
    nk = pl.num_programs(1)

    @pl.when(k == 0)
    def _():
        x = h_ref[...] + nx_ref[...]
        xln_ref[...] = _layer_norm(x, l1s_ref[...], l1b_ref[...])
        acc_ref[...] = jnp.zeros_like(acc_ref)

    xln = xln_ref[...]
    y = _dot(xln, w1_ref[...], (((1,), (1,))), _DEFAULT) + b1_ref[...]
    y = _gelu_exact(y)
    acc_ref[...] += _dot(y, w2_ref[...], (((1,), (1,))), _DEFAULT)

    @pl.when(k == nk - 1)
    def _():
        x2 = xln_ref[...] + acc_ref[...] + b2_ref[...]
        o_ref[...] = _layer_norm(x2, l2s_ref[...], l2b_ref[...])


_F32 = jnp.float32


def _embed(x2, we, be):
    return pl.pallas_call(
        _embed_kernel,
        out_shape=jax.ShapeDtypeStruct((_L, _HID), _F32),
    )(x2, we, be.reshape(1, _HID), jnp.asarray(_PE))


def _term(h, cnt):
    rb, nr = 256, _L // 256
    return pl.pallas_call(
        _term_kernel,
        grid=(nr,),
        in_specs=[
            pl.BlockSpec((rb, _L), lambda r: (r, 0)),
            pl.BlockSpec((_L, _HID), lambda r: (0, 0)),
            pl.BlockSpec((rb, _HID), lambda r: (r, 0)),
        ],
        out_specs=pl.BlockSpec((rb, _NHEAD), lambda r: (r, 0)),
        out_shape=jax.ShapeDtypeStruct((_L, _NHEAD), _F32),
        compiler_params=pltpu.CompilerParams(
            dimension_semantics=("arbitrary",)
        ),
    )(cnt, h, h)


def _m_scores(h4, neg_t):
    rb, nr = 512, _L // 512
    return pl.pallas_call(
        _m_kernel,
        grid=(nr, _NHEAD),
        in_specs=[
            pl.BlockSpec((1, rb, _DH), lambda r, hh: (hh, r, 0)),
            pl.BlockSpec((1, _L, _DH), lambda r, hh: (hh, 0, 0)),
            pl.BlockSpec((_L, rb), lambda r, hh: (0, r)),
        ],
        out_specs=pl.BlockSpec((1, 1, 1, rb), lambda r, hh: (hh, r, 0, 0)),
        out_shape=jax.ShapeDtypeStruct((_NHEAD, nr, 1, rb), _F32),
        compiler_params=pltpu.CompilerParams(
            dimension_semantics=("arbitrary", "arbitrary")
        ),
    )(h4, h4, neg_t)


def _topk(m, term_t):
    return pl.pallas_call(
        _topk_kernel,
        out_shape=jax.ShapeDtypeStruct((_NHEAD, 1, 128), jnp.int32),
    )(m, term_t)


def _attn(h4, ind):
    return pl.pallas_call(
        _attn_kernel,
        grid=(_NHEAD,),
        in_specs=[
            pl.BlockSpec((1, _L, _DH), lambda hh: (hh, 0, 0)),
            pl.BlockSpec((1, 1, 128), lambda hh: (hh, 0, 0)),
        ],
        out_specs=pl.BlockSpec((1, _L, _DH), lambda hh: (hh, 0, 0)),
        out_shape=jax.ShapeDtypeStruct((_NHEAD, _L, _DH), _F32),
        compiler_params=pltpu.CompilerParams(
            dimension_semantics=("arbitrary",)
        ),
    )(h4, ind)


def _ffn(h, nx, l1s, l1b, w1, b1, w2, b2, l2s, l2b):
    rb, nr = 256, _L // 256
    kb, nkk = 512, _DFF // 512
    vec = lambda: pl.BlockSpec((1, _HID), lambda r, k: (0, 0))
    return pl.pallas_call(
        _ffn_kernel,
        grid=(nr, nkk),
        in_specs=[
            pl.BlockSpec((rb, _HID), lambda r, k: (r, 0)),
            pl.BlockSpec((rb, _HID), lambda r, k: (r, 0)),
            vec(), vec(),
            pl.BlockSpec((kb, _HID), lambda r, k: (k, 0)),
            pl.BlockSpec((1, kb), lambda r, k: (0, k)),
            pl.BlockSpec((_HID, kb), lambda r, k: (0, k)),
            vec(),
            vec(), vec(),
        ],
        out_specs=pl.BlockSpec((rb, _HID), lambda r, k: (r, 0)),
        out_shape=jax.ShapeDtypeStruct((_L, _HID), _F32),
        scratch_shapes=[
            pltpu.VMEM((rb, _HID), _F32),
            pltpu.VMEM((rb, _HID), _F32),
        ],
        compiler_params=pltpu.CompilerParams(
            dimension_semantics=("arbitrary", "arbitrary")
        ),
    )(h, nx, l1s.reshape(1, _HID), l1b.reshape(1, _HID), w1,
      b1.reshape(1, _DFF), w2, b2.reshape(1, _HID),
      l2s.reshape(1, _HID), l2b.reshape(1, _HID))


def kernel(x, We, be, ln1_s, ln1_b, W1, b1, W2, b2, ln2_s, ln2_b):
    B = x.shape[0]
    h = _embed(x.reshape(B * _L, -1), We, be)
    for i in range(_NLAYERS):
        h4 = h.reshape(_L, _NHEAD, _DH).transpose(1, 0, 2)
        term_t = _term(h, jnp.asarray(_CNT[i])).T
        m = _m_scores(h4, jnp.asarray(_NEG_T[i]))
        ind = _topk(m.reshape(_NHEAD, _L), term_t)
        nx = _attn(h4, ind).transpose(1, 0, 2).reshape(_L, _HID)
        h = _ffn(h, nx, ln1_s[i], ln1_b[i], W1[i], b1[i], W2[i], b2[i],
                 ln2_s[i], ln2_b[i])
    return h.reshape(B, _L, _HID)


# speedup vs baseline: 4.0616x; 1.2622x over previous
import math

import jax
import jax.numpy as jnp
import numpy as np
from jax.experimental import pallas as pl
from jax.experimental.pallas import tpu as pltpu

_NHEAD = 16
_NLAYERS = 3
_FACTOR = 5
_L = 2048
_DH = 64
_HID = 1024
_DFF = 4096
_NTOP = _FACTOR * int(np.ceil(np.log(_L)))
_HIGHEST = jax.lax.Precision.HIGHEST


def _pos_encoding_np(d_model, max_len=5000):
    position = np.arange(max_len, dtype=np.float32)[:, None]
    div_term = np.exp(
        np.arange(0, d_model, 2, dtype=np.float32) * (-math.log(10000.0) / d_model)
    )
    pe = np.zeros((max_len, d_model), dtype=np.float32)
    pe[:, 0::2] = np.sin(position * div_term)
    pe[:, 1::2] = np.cos(position * div_term)
    return pe


_PE = _pos_encoding_np(_HID)[:_L]


def _rotl(x, d):
    return ((x << np.uint32(d)) | (x >> np.uint32(32 - d))).astype(np.uint32)


def _threefry2x32(k0, k1, x0, x1):
    x0 = x0.astype(np.uint32).copy()
    x1 = x1.astype(np.uint32).copy()
    ks0, ks1 = np.uint32(k0), np.uint32(k1)
    ks2 = np.uint32(ks0 ^ ks1 ^ np.uint32(0x1BD11BDA))
    ks = [ks0, ks1, ks2]
    rot = [[13, 15, 26, 6], [17, 29, 16, 24]]
    x0 += ks0
    x1 += ks1
    for r in range(5):
        for d in rot[r % 2]:
            x0 += x1
            x1 = _rotl(x1, d)
            x1 ^= x0
        x0 += ks[(r + 1) % 3]
        x1 += ks[(r + 2) % 3] + np.uint32(r + 1)
    return x0, x1


def _tf_bits(key, n):
    iot = np.arange(n, dtype=np.uint64)
    x0 = (iot >> np.uint64(32)).astype(np.uint32)
    x1 = (iot & np.uint64(0xFFFFFFFF)).astype(np.uint32)
    o0, o1 = _threefry2x32(key[0], key[1], x0, x1)
    return o0 ^ o1


def _tf_fold_in(key, data):
    d0 = np.uint32((int(data) >> 32) & 0xFFFFFFFF)
    d1 = np.uint32(int(data) & 0xFFFFFFFF)
    o0, o1 = _threefry2x32(key[0], key[1], np.array([d0]), np.array([d1]))
    return (o0[0], o1[0])


def _tf_randint(key, n, span):
    o0, o1 = _threefry2x32(key[0], key[1], np.zeros(2, np.uint32),
                           np.arange(2, dtype=np.uint32))
    hi = _tf_bits((o0[0], o1[0]), n)
    lo = _tf_bits((o0[1], o1[1]), n)
    sp = np.uint32(span)
    mult = np.uint32(((2 ** 16) % span) * ((2 ** 16) % span) % span)
    return ((hi % sp) * mult + lo % sp) % sp


def _sample_constants():
    cnts, negs = [], []
    base = (np.uint32(0), np.uint32(42))
    for i in range(_NLAYERS):
        k = _tf_fold_in(base, i)
        idx = _tf_randint(k, _L * _FACTOR * 8, _L).astype(np.int64)
        idx = idx.reshape(_L, _FACTOR * 8)
        cnt = np.zeros((_L, _L), dtype=np.float32)
        np.add.at(cnt, (np.arange(_L)[:, None], idx), 1.0)
        neg = np.where(cnt > 0.0, 0.0, -1e30).astype(np.float32)
        cnts.append(cnt)
        negs.append(np.ascontiguousarray(neg.T))
    return cnts, negs


_CNT, _NEG_T = _sample_constants()


_DEFAULT = jax.lax.Precision.DEFAULT


def _dot(a, b, dims, prec=_HIGHEST):
    return jax.lax.dot_general(
        a, b, dimension_numbers=(dims, ((), ())),
        precision=prec, preferred_element_type=jnp.float32,
    )


def _dot3(a, b, dims):
    ahi = a.astype(jnp.bfloat16).astype(jnp.float32)
    alo = (a - ahi).astype(jnp.bfloat16)
    bhi = b.astype(jnp.bfloat16).astype(jnp.float32)
    blo = (b - bhi).astype(jnp.bfloat16)
    ah = ahi.astype(jnp.bfloat16)
    bh = bhi.astype(jnp.bfloat16)
    return (_dot(ah, bh, dims, _DEFAULT)
            + _dot(ah, blo, dims, _DEFAULT)
            + _dot(alo, bh, dims, _DEFAULT))


def _embed_kernel(x_ref, we_ref, be_ref, pe_ref, o_ref):
    h = _dot(x_ref[...], we_ref[...], (((1,), (1,))), _DEFAULT)
    o_ref[...] = h + be_ref[...] + pe_ref[...]


def _term_kernel(cnt_ref, hfull_ref, hblk_ref, t_ref):
    ks = _dot(cnt_ref[...], hfull_ref[...], (((1,), (0,))), _DEFAULT)
    prod = ks * hblk_ref[...]
    rb = prod.shape[0]
    t_ref[...] = jnp.sum(prod.reshape(rb, _NHEAD, _DH), axis=2)


def _m_kernel(q_ref, k_ref, neg_ref, m_ref):
    s = _dot3(k_ref[0], q_ref[0], (((1,), (1,))))
    m_ref[0, 0, 0, :] = jnp.max(s + neg_ref[...], axis=0)


def _topk_kernel(m_ref, t_ref, o_ref):
    mc = m_ref[...] - t_ref[...] * (1.0 / _L)
    iota = jax.lax.broadcasted_iota(jnp.int32, (_NHEAD, _L), 1)
    acc_iota = jax.lax.broadcasted_iota(jnp.int32, (_NHEAD, 128), 1)

    def body(u, carry):
        mcur, acc = carry
        mval = jnp.max(mcur, axis=1, keepdims=True)
        sel = mcur == mval
        idxv = jnp.min(jnp.where(sel, iota, _L), axis=1, keepdims=True)
        acc = jnp.where(acc_iota == u, idxv, acc)
        mcur = jnp.where(iota == idxv, -jnp.inf, mcur)
        return mcur, acc

    _, acc = jax.lax.fori_loop(
        0, _NTOP, body, (mc, jnp.full((_NHEAD, 128), -1, jnp.int32))
    )
    o_ref[:, 0, :] = acc


def _attn_kernel(h_ref, ind_ref, o_ref):
    q = h_ref[0]
    ind = ind_ref[0][:, :64]
    rows = jax.lax.broadcasted_iota(jnp.int32, (_L, 64), 0)
    p = (rows == ind).astype(jnp.float32)
    qr = _dot(p, q, (((0,), (0,))), _DEFAULT)
    scores = _dot(qr, q, (((1,), (1,))), _DEFAULT) * (1.0 / math.sqrt(_DH))
    mx = jnp.max(scores, axis=1, keepdims=True)
    e = jnp.exp(scores - mx)
    sm = e / jnp.sum(e, axis=1, keepdims=True)
    upd = _dot(sm, q, (((1,), (0,))), _DEFAULT)
    meanv = jnp.mean(q, axis=0, keepdims=True)
    o_ref[0, :, :] = meanv + _dot(p, upd - meanv, (((1,), (0,))), _DEFAULT)


def _erf(a):
    s = jnp.sign(a)
    x = jnp.abs(a)
    t = 1.0 / (1.0 + 0.3275911 * x)
    p = ((((1.061405429 * t - 1.453152027) * t + 1.421413741) * t
          - 0.284496736) * t + 0.254829592) * t
    return s * (1.0 - p * jnp.exp(-x * x))


def _gelu_exact(x):
    return 0.5 * x * (1.0 + _erf(x * np.float32(1.0 / math.sqrt(2.0))))


def _layer_norm(x, scale, bias):
    mu = jnp.mean(x, axis=1, keepdims=True)
    var = jnp.mean((x - mu) ** 2, axis=1, keepdims=True)
    return (x - mu) / jnp.sqrt(var + 1e-5) * scale + bias


def _ffn_kernel(h_ref, nx_ref, l1s_ref, l1b_ref, w1_ref, b1_ref, w2_ref,
                b2_ref, l2s_ref, l2b_ref, o_ref, xln_ref, acc_ref):
    k = pl.program_id(1)
    nk = pl.num_programs(1)

    @pl.when(k == 0)
    def _():
        x = h_ref[...] + nx_ref[...]
        xln_ref[...] = _layer_norm(x, l1s_ref[...], l1b_ref[...])
        acc_ref[...] = jnp.zeros_like(acc_ref)

    xln = xln_ref[...]
    y = _dot(xln, w1_ref[...], (((1,), (1,))), _DEFAULT) + b1_ref[...]
    y = _gelu_exact(y)
    acc_ref[...] += _dot(y, w2_ref[...], (((1,), (1,))), _DEFAULT)

    @pl.when(k == nk - 1)
    def _():
        x2 = xln_ref[...] + acc_ref[...] + b2_ref[...]
        o_ref[...] = _layer_norm(x2, l2s_ref[...], l2b_ref[...])


_F32 = jnp.float32


def _embed(x2, we, be):
    return pl.pallas_call(
        _embed_kernel,
        out_shape=jax.ShapeDtypeStruct((_L, _HID), _F32),
    )(x2, we, be.reshape(1, _HID), jnp.asarray(_PE))


def _term(h, cnt):
    rb, nr = 256, _L // 256
    return pl.pallas_call(
        _term_kernel,
        grid=(nr,),
        in_specs=[
            pl.BlockSpec((rb, _L), lambda r: (r, 0)),
            pl.BlockSpec((_L, _HID), lambda r: (0, 0)),
            pl.BlockSpec((rb, _HID), lambda r: (r, 0)),
        ],
        out_specs=pl.BlockSpec((rb, _NHEAD), lambda r: (r, 0)),
        out_shape=jax.ShapeDtypeStruct((_L, _NHEAD), _F32),
        compiler_params=pltpu.CompilerParams(
            dimension_semantics=("arbitrary",)
        ),
    )(cnt, h, h)


def _m_scores(h4, neg_t):
    rb, nr = 1024, _L // 1024
    return pl.pallas_call(
        _m_kernel,
        grid=(nr, _NHEAD),
        in_specs=[
            pl.BlockSpec((1, rb, _DH), lambda r, hh: (hh, r, 0)),
            pl.BlockSpec((1, _L, _DH), lambda r, hh: (hh, 0, 0)),
            pl.BlockSpec((_L, rb), lambda r, hh: (0, r)),
        ],
        out_specs=pl.BlockSpec((1, 1, 1, rb), lambda r, hh: (hh, r, 0, 0)),
        out_shape=jax.ShapeDtypeStruct((_NHEAD, nr, 1, rb), _F32),
        compiler_params=pltpu.CompilerParams(
            dimension_semantics=("arbitrary", "arbitrary")
        ),
    )(h4, h4, neg_t)


def _topk(m, term_t):
    return pl.pallas_call(
        _topk_kernel,
        out_shape=jax.ShapeDtypeStruct((_NHEAD, 1, 128), jnp.int32),
    )(m, term_t)


def _attn(h4, ind):
    return pl.pallas_call(
        _attn_kernel,
        grid=(_NHEAD,),
        in_specs=[
            pl.BlockSpec((1, _L, _DH), lambda hh: (hh, 0, 0)),
            pl.BlockSpec((1, 1, 128), lambda hh: (hh, 0, 0)),
        ],
        out_specs=pl.BlockSpec((1, _L, _DH), lambda hh: (hh, 0, 0)),
        out_shape=jax.ShapeDtypeStruct((_NHEAD, _L, _DH), _F32),
        compiler_params=pltpu.CompilerParams(
            dimension_semantics=("arbitrary",)
        ),
    )(h4, ind)


def _ffn(h, nx, l1s, l1b, w1, b1, w2, b2, l2s, l2b):
    rb, nr = 512, _L // 512
    kb, nkk = 512, _DFF // 512
    vec = lambda: pl.BlockSpec((1, _HID), lambda r, k: (0, 0))
    return pl.pallas_call(
        _ffn_kernel,
        grid=(nr, nkk),
        in_specs=[
            pl.BlockSpec((rb, _HID), lambda r, k: (r, 0)),
            pl.BlockSpec((rb, _HID), lambda r, k: (r, 0)),
            vec(), vec(),
            pl.BlockSpec((kb, _HID), lambda r, k: (k, 0)),
            pl.BlockSpec((1, kb), lambda r, k: (0, k)),
            pl.BlockSpec((_HID, kb), lambda r, k: (0, k)),
            vec(),
            vec(), vec(),
        ],
        out_specs=pl.BlockSpec((rb, _HID), lambda r, k: (r, 0)),
        out_shape=jax.ShapeDtypeStruct((_L, _HID), _F32),
        scratch_shapes=[
            pltpu.VMEM((rb, _HID), _F32),
            pltpu.VMEM((rb, _HID), _F32),
        ],
        compiler_params=pltpu.CompilerParams(
            dimension_semantics=("arbitrary", "arbitrary")
        ),
    )(h, nx, l1s.reshape(1, _HID), l1b.reshape(1, _HID), w1,
      b1.reshape(1, _DFF), w2, b2.reshape(1, _HID),
      l2s.reshape(1, _HID), l2b.reshape(1, _HID))


def kernel(x, We, be, ln1_s, ln1_b, W1, b1, W2, b2, ln2_s, ln2_b):
    B = x.shape[0]
    h = _embed(x.reshape(B * _L, -1), We, be)
    for i in range(_NLAYERS):
        h4 = h.reshape(_L, _NHEAD, _DH).transpose(1, 0, 2)
        term_t = _term(h, jnp.asarray(_CNT[i])).T
        m = _m_scores(h4, jnp.asarray(_NEG_T[i]))
        ind = _topk(m.reshape(_NHEAD, _L), term_t)
        nx = _attn(h4, ind).transpose(1, 0, 2).reshape(_L, _HID)
        h = _ffn(h, nx, ln1_s[i], ln1_b[i], W1[i], b1[i], W2[i], b2[i],
                 ln2_s[i], ln2_b[i])
    return h.reshape(B, _L, _HID)


# speedup vs baseline: 4.5731x; 1.1259x over previous
import math

import jax
import jax.numpy as jnp
import numpy as np
from jax.experimental import pallas as pl
from jax.experimental.pallas import tpu as pltpu

_NHEAD = 16
_NLAYERS = 3
_FACTOR = 5
_L = 2048
_DH = 64
_HID = 1024
_DFF = 4096
_NTOP = _FACTOR * int(np.ceil(np.log(_L)))
_HIGHEST = jax.lax.Precision.HIGHEST


def _pos_encoding_np(d_model, max_len=5000):
    position = np.arange(max_len, dtype=np.float32)[:, None]
    div_term = np.exp(
        np.arange(0, d_model, 2, dtype=np.float32) * (-math.log(10000.0) / d_model)
    )
    pe = np.zeros((max_len, d_model), dtype=np.float32)
    pe[:, 0::2] = np.sin(position * div_term)
    pe[:, 1::2] = np.cos(position * div_term)
    return pe


_PE = _pos_encoding_np(_HID)[:_L]


def _rotl(x, d):
    return ((x << np.uint32(d)) | (x >> np.uint32(32 - d))).astype(np.uint32)


def _threefry2x32(k0, k1, x0, x1):
    x0 = x0.astype(np.uint32).copy()
    x1 = x1.astype(np.uint32).copy()
    ks0, ks1 = np.uint32(k0), np.uint32(k1)
    ks2 = np.uint32(ks0 ^ ks1 ^ np.uint32(0x1BD11BDA))
    ks = [ks0, ks1, ks2]
    rot = [[13, 15, 26, 6], [17, 29, 16, 24]]
    x0 += ks0
    x1 += ks1
    for r in range(5):
        for d in rot[r % 2]:
            x0 += x1
            x1 = _rotl(x1, d)
            x1 ^= x0
        x0 += ks[(r + 1) % 3]
        x1 += ks[(r + 2) % 3] + np.uint32(r + 1)
    return x0, x1


def _tf_bits(key, n):
    iot = np.arange(n, dtype=np.uint64)
    x0 = (iot >> np.uint64(32)).astype(np.uint32)
    x1 = (iot & np.uint64(0xFFFFFFFF)).astype(np.uint32)
    o0, o1 = _threefry2x32(key[0], key[1], x0, x1)
    return o0 ^ o1


def _tf_fold_in(key, data):
    d0 = np.uint32((int(data) >> 32) & 0xFFFFFFFF)
    d1 = np.uint32(int(data) & 0xFFFFFFFF)
    o0, o1 = _threefry2x32(key[0], key[1], np.array([d0]), np.array([d1]))
    return (o0[0], o1[0])


def _tf_randint(key, n, span):
    o0, o1 = _threefry2x32(key[0], key[1], np.zeros(2, np.uint32),
                           np.arange(2, dtype=np.uint32))
    hi = _tf_bits((o0[0], o1[0]), n)
    lo = _tf_bits((o0[1], o1[1]), n)
    sp = np.uint32(span)
    mult = np.uint32(((2 ** 16) % span) * ((2 ** 16) % span) % span)
    return ((hi % sp) * mult + lo % sp) % sp


def _sample_constants():
    cnts, negs = [], []
    base = (np.uint32(0), np.uint32(42))
    for i in range(_NLAYERS):
        k = _tf_fold_in(base, i)
        idx = _tf_randint(k, _L * _FACTOR * 8, _L).astype(np.int64)
        idx = idx.reshape(_L, _FACTOR * 8)
        cnt = np.zeros((_L, _L), dtype=np.float32)
        np.add.at(cnt, (np.arange(_L)[:, None], idx), 1.0)
        neg = np.where(cnt > 0.0, 0.0, -1e30).astype(np.float32)
        cnts.append(cnt)
        negs.append(np.ascontiguousarray(neg.T))
    return cnts, negs


_CNT, _NEG_T = _sample_constants()


_DEFAULT = jax.lax.Precision.DEFAULT


def _dot(a, b, dims, prec=_HIGHEST):
    return jax.lax.dot_general(
        a, b, dimension_numbers=(dims, ((), ())),
        precision=prec, preferred_element_type=jnp.float32,
    )


def _dot3(a, b, dims):
    ahi = a.astype(jnp.bfloat16).astype(jnp.float32)
    alo = (a - ahi).astype(jnp.bfloat16)
    bhi = b.astype(jnp.bfloat16).astype(jnp.float32)
    blo = (b - bhi).astype(jnp.bfloat16)
    ah = ahi.astype(jnp.bfloat16)
    bh = bhi.astype(jnp.bfloat16)
    return (_dot(ah, bh, dims, _DEFAULT)
            + _dot(ah, blo, dims, _DEFAULT)
            + _dot(alo, bh, dims, _DEFAULT))


def _embed_kernel(x_ref, we_ref, be_ref, pe_ref, o_ref):
    h = _dot(x_ref[...], we_ref[...], (((1,), (1,))), _DEFAULT)
    o_ref[...] = h + be_ref[...] + pe_ref[...]


def _term_kernel(cnt_ref, hfull_ref, hblk_ref, t_ref):
    ks = _dot(cnt_ref[...], hfull_ref[...], (((1,), (0,))), _DEFAULT)
    prod = ks * hblk_ref[...]
    rb = prod.shape[0]
    t_ref[...] = jnp.sum(prod.reshape(rb, _NHEAD, _DH), axis=2)


def _m_kernel(q_ref, k_ref, neg_ref, m_ref):
    neg = neg_ref[...]
    for t in range(2):
        q = q_ref[:, t * _DH:(t + 1) * _DH]
        k = k_ref[:, t * _DH:(t + 1) * _DH]
        s = _dot3(k, q, (((1,), (1,))))
        m_ref[t, 0, 0, :] = jnp.max(s + neg, axis=0)


def _topk_kernel(m_ref, t_ref, o_ref):
    mc = m_ref[...] - t_ref[...] * (1.0 / _L)
    iota = jax.lax.broadcasted_iota(jnp.int32, (_NHEAD, _L), 1)
    acc_iota = jax.lax.broadcasted_iota(jnp.int32, (_NHEAD, 128), 1)

    def body(u, carry):
        mcur, acc = carry
        mval = jnp.max(mcur, axis=1, keepdims=True)
        sel = mcur == mval
        idxv = jnp.min(jnp.where(sel, iota, _L), axis=1, keepdims=True)
        acc = jnp.where(acc_iota == u, idxv, acc)
        mcur = jnp.where(iota == idxv, -jnp.inf, mcur)
        return mcur, acc

    _, acc = jax.lax.fori_loop(
        0, _NTOP, body, (mc, jnp.full((_NHEAD, 128), -1, jnp.int32))
    )
    o_ref[:, 0, :] = acc


def _attn_kernel(h_ref, ind_ref, o_ref):
    rows = jax.lax.broadcasted_iota(jnp.int32, (_L, 64), 0)
    outs = []
    for t in range(2):
        q = h_ref[:, t * _DH:(t + 1) * _DH]
        ind = ind_ref[t][:, :64]
        p = (rows == ind).astype(jnp.float32)
        qr = _dot(p, q, (((0,), (0,))), _DEFAULT)
        scores = _dot(qr, q, (((1,), (1,))), _DEFAULT) * (1.0 / math.sqrt(_DH))
        mx = jnp.max(scores, axis=1, keepdims=True)
        e = jnp.exp(scores - mx)
        sm = e / jnp.sum(e, axis=1, keepdims=True)
        upd = _dot(sm, q, (((1,), (0,))), _DEFAULT)
        meanv = jnp.mean(q, axis=0, keepdims=True)
        outs.append(meanv + _dot(p, upd - meanv, (((1,), (0,))), _DEFAULT))
    o_ref[...] = jnp.concatenate(outs, axis=1)


def _erf(a):
    s = jnp.sign(a)
    x = jnp.abs(a)
    t = 1.0 / (1.0 + 0.3275911 * x)
    p = ((((1.061405429 * t - 1.453152027) * t + 1.421413741) * t
          - 0.284496736) * t + 0.254829592) * t
    return s * (1.0 - p * jnp.exp(-x * x))


def _gelu_exact(x):
    return 0.5 * x * (1.0 + _erf(x * np.float32(1.0 / math.sqrt(2.0))))


def _layer_norm(x, scale, bias):
    mu = jnp.mean(x, axis=1, keepdims=True)
    var = jnp.mean((x - mu) ** 2, axis=1, keepdims=True)
    return (x - mu) / jnp.sqrt(var + 1e-5) * scale + bias


def _ffn_kernel(h_ref, nx_ref, l1s_ref, l1b_ref, w1_ref, b1_ref, w2_ref,
                b2_ref, l2s_ref, l2b_ref, o_ref, xln_ref, acc_ref):
    k = pl.program_id(1)
    nk = pl.num_programs(1)

    @pl.when(k == 0)
    def _():
        x = h_ref[...] + nx_ref[...]
        xln_ref[...] = _layer_norm(x, l1s_ref[...], l1b_ref[...])
        acc_ref[...] = jnp.zeros_like(acc_ref)

    xln = xln_ref[...]
    y = _dot(xln, w1_ref[...], (((1,), (1,))), _DEFAULT) + b1_ref[...]
    y = _gelu_exact(y)
    acc_ref[...] += _dot(y, w2_ref[...], (((1,), (1,))), _DEFAULT)

    @pl.when(k == nk - 1)
    def _():
        x2 = xln_ref[...] + acc_ref[...] + b2_ref[...]
        o_ref[...] = _layer_norm(x2, l2s_ref[...], l2b_ref[...])


_F32 = jnp.float32


def _embed(x2, we, be):
    return pl.pallas_call(
        _embed_kernel,
        out_shape=jax.ShapeDtypeStruct((_L, _HID), _F32),
    )(x2, we, be.reshape(1, _HID), jnp.asarray(_PE))


def _term(h, cnt):
    rb, nr = 256, _L // 256
    return pl.pallas_call(
        _term_kernel,
        grid=(nr,),
        in_specs=[
            pl.BlockSpec((rb, _L), lambda r: (r, 0)),
            pl.BlockSpec((_L, _HID), lambda r: (0, 0)),
            pl.BlockSpec((rb, _HID), lambda r: (r, 0)),
        ],
        out_specs=pl.BlockSpec((rb, _NHEAD), lambda r: (r, 0)),
        out_shape=jax.ShapeDtypeStruct((_L, _NHEAD), _F32),
        compiler_params=pltpu.CompilerParams(
            dimension_semantics=("arbitrary",)
        ),
    )(cnt, h, h)


def _m_scores(h, neg_t):
    rb, nr = 512, _L // 512
    return pl.pallas_call(
        _m_kernel,
        grid=(nr, _NHEAD // 2),
        in_specs=[
            pl.BlockSpec((rb, 128), lambda r, hh: (r, hh)),
            pl.BlockSpec((_L, 128), lambda r, hh: (0, hh)),
            pl.BlockSpec((_L, rb), lambda r, hh: (0, r)),
        ],
        out_specs=pl.BlockSpec((2, 1, 1, rb), lambda r, hh: (hh, r, 0, 0)),
        out_shape=jax.ShapeDtypeStruct((_NHEAD, nr, 1, rb), _F32),
        compiler_params=pltpu.CompilerParams(
            dimension_semantics=("arbitrary", "arbitrary")
        ),
    )(h, h, neg_t)


def _topk(m, term_t):
    return pl.pallas_call(
        _topk_kernel,
        out_shape=jax.ShapeDtypeStruct((_NHEAD, 1, 128), jnp.int32),
    )(m, term_t)


def _attn(h, ind):
    return pl.pallas_call(
        _attn_kernel,
        grid=(_NHEAD // 2,),
        in_specs=[
            pl.BlockSpec((_L, 128), lambda hh: (0, hh)),
            pl.BlockSpec((2, 1, 128), lambda hh: (hh, 0, 0)),
        ],
        out_specs=pl.BlockSpec((_L, 128), lambda hh: (0, hh)),
        out_shape=jax.ShapeDtypeStruct((_L, _HID), _F32),
        compiler_params=pltpu.CompilerParams(
            dimension_semantics=("arbitrary",)
        ),
    )(h, ind)


def _ffn(h, nx, l1s, l1b, w1, b1, w2, b2, l2s, l2b):
    rb, nr = 512, _L // 512
    kb, nkk = 512, _DFF // 512
    vec = lambda: pl.BlockSpec((1, _HID), lambda r, k: (0, 0))
    return pl.pallas_call(
        _ffn_kernel,
        grid=(nr, nkk),
        in_specs=[
            pl.BlockSpec((rb, _HID), lambda r, k: (r, 0)),
            pl.BlockSpec((rb, _HID), lambda r, k: (r, 0)),
            vec(), vec(),
            pl.BlockSpec((kb, _HID), lambda r, k: (k, 0)),
            pl.BlockSpec((1, kb), lambda r, k: (0, k)),
            pl.BlockSpec((_HID, kb), lambda r, k: (0, k)),
            vec(),
            vec(), vec(),
        ],
        out_specs=pl.BlockSpec((rb, _HID), lambda r, k: (r, 0)),
        out_shape=jax.ShapeDtypeStruct((_L, _HID), _F32),
        scratch_shapes=[
            pltpu.VMEM((rb, _HID), _F32),
            pltpu.VMEM((rb, _HID), _F32),
        ],
        compiler_params=pltpu.CompilerParams(
            dimension_semantics=("arbitrary", "arbitrary")
        ),
    )(h, nx, l1s.reshape(1, _HID), l1b.reshape(1, _HID), w1,
      b1.reshape(1, _DFF), w2, b2.reshape(1, _HID),
      l2s.reshape(1, _HID), l2b.reshape(1, _HID))


def kernel(x, We, be, ln1_s, ln1_b, W1, b1, W2, b2, ln2_s, ln2_b):
    B = x.shape[0]
    h = _embed(x.reshape(B * _L, -1), We, be)
    for i in range(_NLAYERS):
        term_t = _term(h, jnp.asarray(_CNT[i])).T
        m = _m_scores(h, jnp.asarray(_NEG_T[i]))
        ind = _topk(m.reshape(_NHEAD, _L), term_t)
        nx = _attn(h, ind)
        h = _ffn(h, nx, ln1_s[i], ln1_b[i], W1[i], b1[i], W2[i], b2[i],
                 ln2_s[i], ln2_b[i])
    return h.reshape(B, _L, _HID)
